# Initial kernel scaffold; baseline (speedup 1.0000x reference)
import jax
import jax.numpy as jnp
from jax.experimental import pallas as pl


def kernel(x, edge_index, hop_W0, hop_b0, hop_g0, hop_be0, hop_W1, hop_b1, hop_g1, hop_be1, imp_W1, imp_b1, imp_W2, imp_b2):
    raise NotImplementedError("write your pallas kernel here")


# speedup vs baseline: 35.3138x; 35.3138x over previous
import functools

import jax
import jax.numpy as jnp
from jax import lax
from jax.experimental import pallas as pl
from jax.experimental.pallas import tpu as pltpu
from jax.experimental.pallas import tpu_sc as plsc

N_NODES = 10000
N_PAD = 10240
HIDDEN = 128
TOP_K = 10
N_EDGES = 320000
NC = 2
NS = 16
NW = NC * NS
EPW = N_EDGES // NW


def _sc_mask_body(edge_hbm, out_hbm, idx_v, mask_v):
    wid = lax.axis_index("s") * NC + lax.axis_index("c")
    base = wid * EPW
    pltpu.sync_copy(edge_hbm.at[pl.ds(base, EPW)], idx_v)

    zeros16 = jnp.zeros((16,), jnp.float32)

    def zero_body(i, carry):
        mask_v[pl.ds(i * 16, 16)] = zeros16
        return carry

    lax.fori_loop(0, N_PAD // 16, zero_body, 0)

    ones16 = jnp.ones((16,), jnp.float32)

    def scatter_body(i, carry):
        idx = idx_v[pl.ds(i * 16, 16)]
        plsc.store_scatter(mask_v, [idx], ones16)
        return carry

    lax.fori_loop(0, EPW // 16, scatter_body, 0)
    pltpu.sync_copy(mask_v, out_hbm.at[wid])


@functools.cache
def _sc_mask():
    return pl.kernel(
        _sc_mask_body,
        mesh=plsc.VectorSubcoreMesh(
            core_axis_name="c", subcore_axis_name="s",
            num_cores=NC, num_subcores=NS),
        out_type=jax.ShapeDtypeStruct((NW, N_PAD), jnp.float32),
        scratch_types=[
            pltpu.VMEM((EPW,), jnp.int32),
            pltpu.VMEM((N_PAD,), jnp.float32),
        ],
        compiler_params=pltpu.CompilerParams(needs_layout_passes=False),
    )


def _tc_body(xT_ref, mask_ref, w1T_ref, b1_ref, w2c_ref, b2_ref,
             w0T_ref, b0_ref, g0_ref, be0_ref, sel_ref, hT_ref):
    xT = xT_ref[...]
    neg_inf = jnp.float32(-jnp.inf)

    h1 = jnp.dot(w1T_ref[...], xT, preferred_element_type=jnp.float32)
    h1 = jnp.maximum(h1 + b1_ref[...], 0.0)
    impT = jnp.sum(h1 * w2c_ref[...], axis=0, keepdims=True) + b2_ref[...]

    msum = jnp.sum(mask_ref[...], axis=0, keepdims=True)
    score = jnp.where(msum > 0.0, impT, neg_inf)

    idxs = lax.broadcasted_iota(jnp.int32, (1, N_PAD), 1)
    avail = idxs < N_NODES
    sels = []
    for _ in range(TOP_K):
        cand = jnp.where(avail, score, neg_inf)
        m = jnp.max(cand)
        eq = (cand == m) & avail
        sel = jnp.min(jnp.where(eq, idxs, N_PAD))
        sels.append(sel)
        avail = avail & (idxs != sel)

    lane16 = lax.broadcasted_iota(jnp.int32, (1, 16), 1)
    selrow = jnp.full((1, 16), -1, jnp.int32)
    for k in range(TOP_K):
        selrow = jnp.where(lane16 == k, sels[k], selrow)

    rowiota = lax.broadcasted_iota(jnp.int32, (N_PAD, 16), 0)
    onehotT = (rowiota == selrow).astype(jnp.float32)
    tgtT = jnp.dot(xT, onehotT, preferred_element_type=jnp.float32)

    meanT = jnp.sum(xT, axis=1, keepdims=True) * (1.0 / N_NODES)
    srcT = jnp.broadcast_to(meanT, (HIDDEN, 16))
    combinedT = jnp.concatenate([srcT, tgtT], axis=0)

    zT = jnp.dot(w0T_ref[...], combinedT,
                 preferred_element_type=jnp.float32) + b0_ref[...]
    mu = jnp.mean(zT, axis=0, keepdims=True)
    var = jnp.mean((zT - mu) ** 2, axis=0, keepdims=True)
    hT = (zT - mu) / jnp.sqrt(var + 1e-5) * g0_ref[...] + be0_ref[...]
    hT_ref[...] = jnp.maximum(hT, 0.0)

    r8 = lax.broadcasted_iota(jnp.int32, (8, 128), 0)
    c128 = lax.broadcasted_iota(jnp.int32, (8, 128), 1)
    selmat = jnp.zeros((8, 128), jnp.int32)
    for k in range(TOP_K):
        selmat = jnp.where((r8 == 0) & (c128 == k), sels[k], selmat)
    sel_ref[...] = selmat


_tc_call = pl.pallas_call(
    _tc_body,
    out_shape=[
        jax.ShapeDtypeStruct((8, 128), jnp.int32),
        jax.ShapeDtypeStruct((HIDDEN, 16), jnp.float32),
    ],
)


def kernel(x, edge_index, hop_W0, hop_b0, hop_g0, hop_be0,
           hop_W1, hop_b1, hop_g1, hop_be1, imp_W1, imp_b1, imp_W2, imp_b2):
    edge_dst = edge_index[1].astype(jnp.int32)
    mask32 = _sc_mask()(edge_dst)

    xT = jnp.pad(x.astype(jnp.float32).T, ((0, 0), (0, N_PAD - N_NODES)))
    sel_mat, hT = _tc_call(
        xT,
        mask32,
        imp_W1.T,
        imp_b1.reshape(-1, 1),
        imp_W2.reshape(-1, 1),
        imp_b2.reshape(1, 1),
        hop_W0.T,
        hop_b0.reshape(-1, 1),
        hop_g0.reshape(-1, 1),
        hop_be0.reshape(-1, 1),
    )
    selected = sel_mat[0, :TOP_K]
    h = hT.T[:TOP_K]
    return (selected, h)


# speedup vs baseline: 37.9896x; 1.0758x over previous
import functools

import jax
import jax.numpy as jnp
from jax import lax
from jax.experimental import pallas as pl
from jax.experimental.pallas import tpu as pltpu
from jax.experimental.pallas import tpu_sc as plsc

N_NODES = 10000
HIDDEN = 128
TOP_K = 10
N_EDGES = 320000
NC = 2
NS = 16
NW = NC * NS
EPW = N_EDGES // NW


def _sc_mask_body(edge_hbm, out_hbm, idx_v, mask_v):
    wid = lax.axis_index("s") * NC + lax.axis_index("c")
    base = wid * EPW
    pltpu.sync_copy(edge_hbm.at[pl.ds(base, EPW)], idx_v)

    zeros16 = jnp.zeros((16,), jnp.float32)

    def zero_body(i, carry):
        mask_v[pl.ds(i * 16, 16)] = zeros16
        return carry

    lax.fori_loop(0, N_NODES // 16, zero_body, 0)

    ones16 = jnp.ones((16,), jnp.float32)

    def scatter_body(i, carry):
        idx = idx_v[pl.ds(i * 16, 16)]
        plsc.store_scatter(mask_v, [idx], ones16)
        return carry

    lax.fori_loop(0, EPW // 16, scatter_body, 0)
    pltpu.sync_copy(mask_v, out_hbm.at[wid])


@functools.cache
def _sc_mask():
    return pl.kernel(
        _sc_mask_body,
        mesh=plsc.VectorSubcoreMesh(
            core_axis_name="c", subcore_axis_name="s",
            num_cores=NC, num_subcores=NS),
        out_type=jax.ShapeDtypeStruct((NW, N_NODES), jnp.float32),
        scratch_types=[
            pltpu.VMEM((EPW,), jnp.int32),
            pltpu.VMEM((N_NODES,), jnp.float32),
        ],
        compiler_params=pltpu.CompilerParams(needs_layout_passes=False),
    )


def _tc_body(x_ref, mask_ref, w1_ref, b1_ref, w2r_ref, b2_ref,
             w0_ref, b0_ref, g0_ref, be0_ref, sel_ref, h_ref):
    x = x_ref[...]
    neg_inf = jnp.float32(-jnp.inf)

    h1 = jnp.dot(x, w1_ref[...], preferred_element_type=jnp.float32)
    h1 = jnp.maximum(h1 + b1_ref[...], 0.0)
    impT = lax.dot_general(w2r_ref[...], h1, (((1,), (1,)), ((), ())),
                           preferred_element_type=jnp.float32) + b2_ref[...]

    msum = jnp.sum(mask_ref[...], axis=0, keepdims=True)
    score = jnp.where(msum > 0.0, impT, neg_inf)

    idxs = lax.broadcasted_iota(jnp.int32, (1, N_NODES), 1)
    avail = idxs >= 0
    sels = []
    for _ in range(TOP_K):
        cand = jnp.where(avail, score, neg_inf)
        m = jnp.max(cand)
        eq = (cand == m) & avail
        sel = jnp.min(jnp.where(eq, idxs, N_NODES))
        sels.append(sel)
        avail = avail & (idxs != sel)

    row16 = lax.broadcasted_iota(jnp.int32, (16, 1), 0)
    selcol = jnp.full((16, 1), -1, jnp.int32)
    for k in range(TOP_K):
        selcol = jnp.where(row16 == k, sels[k], selcol)

    coliota = lax.broadcasted_iota(jnp.int32, (16, N_NODES), 1)
    onehot = (coliota == selcol).astype(jnp.float32)
    tgt = jnp.dot(onehot, x, preferred_element_type=jnp.float32)

    mean = jnp.sum(x, axis=0, keepdims=True) * (1.0 / N_NODES)
    src = jnp.broadcast_to(mean, (16, HIDDEN))
    combined = jnp.concatenate([src, tgt], axis=1)

    z = jnp.dot(combined, w0_ref[...],
                preferred_element_type=jnp.float32) + b0_ref[...]
    mu = jnp.mean(z, axis=1, keepdims=True)
    var = jnp.mean((z - mu) ** 2, axis=1, keepdims=True)
    h = (z - mu) / jnp.sqrt(var + 1e-5) * g0_ref[...] + be0_ref[...]
    h_ref[...] = jnp.maximum(h, 0.0)

    r8 = lax.broadcasted_iota(jnp.int32, (8, 128), 0)
    c128 = lax.broadcasted_iota(jnp.int32, (8, 128), 1)
    selmat = jnp.zeros((8, 128), jnp.int32)
    for k in range(TOP_K):
        selmat = jnp.where((r8 == 0) & (c128 == k), sels[k], selmat)
    sel_ref[...] = selmat


_tc_call = pl.pallas_call(
    _tc_body,
    out_shape=[
        jax.ShapeDtypeStruct((8, 128), jnp.int32),
        jax.ShapeDtypeStruct((16, HIDDEN), jnp.float32),
    ],
)


def kernel(x, edge_index, hop_W0, hop_b0, hop_g0, hop_be0,
           hop_W1, hop_b1, hop_g1, hop_be1, imp_W1, imp_b1, imp_W2, imp_b2):
    edge_dst = edge_index[1].astype(jnp.int32)
    mask32 = _sc_mask()(edge_dst)

    sel_mat, h16 = _tc_call(
        x.astype(jnp.float32),
        mask32,
        imp_W1,
        imp_b1.reshape(1, -1),
        imp_W2.reshape(1, -1),
        imp_b2.reshape(1, 1),
        hop_W0,
        hop_b0.reshape(1, -1),
        hop_g0.reshape(1, -1),
        hop_be0.reshape(1, -1),
    )
    selected = sel_mat[0, :TOP_K]
    h = h16[:TOP_K]
    return (selected, h)


# speedup vs baseline: 40.8938x; 1.0764x over previous
import functools

import jax
import jax.numpy as jnp
from jax import lax
from jax.experimental import pallas as pl
from jax.experimental.pallas import tpu as pltpu
from jax.experimental.pallas import tpu_sc as plsc

N_NODES = 10000
HIDDEN = 128
TOP_K = 10
N_EDGES = 320000
NC = 2
NS = 16
NW = NC * NS
EPW = N_EDGES // NW


_UNROLL = 8


def _sc_mask_body(edge_hbm, out_hbm, idx_v, mask_v):
    wid = lax.axis_index("s") * NC + lax.axis_index("c")
    base = wid * EPW
    pltpu.sync_copy(edge_hbm.at[pl.ds(base, EPW)], idx_v)

    zeros16 = jnp.zeros((16,), jnp.float32)

    def zero_body(i, carry):
        for u in range(_UNROLL):
            mask_v[pl.ds((i * _UNROLL + u) * 16, 16)] = zeros16
        return carry

    lax.fori_loop(0, N_NODES // (16 * _UNROLL), zero_body, 0)
    mask_v[pl.ds(N_NODES - 16, 16)] = zeros16

    ones16 = jnp.ones((16,), jnp.float32)

    def scatter_body(i, carry):
        for u in range(_UNROLL):
            idx = idx_v[pl.ds((i * _UNROLL + u) * 16, 16)]
            plsc.store_scatter(mask_v, [idx], ones16)
        return carry

    lax.fori_loop(0, EPW // (16 * _UNROLL), scatter_body, 0)
    idx = idx_v[pl.ds(EPW - 16, 16)]
    plsc.store_scatter(mask_v, [idx], ones16)
    pltpu.sync_copy(mask_v, out_hbm.at[wid])


@functools.cache
def _sc_mask():
    return pl.kernel(
        _sc_mask_body,
        mesh=plsc.VectorSubcoreMesh(
            core_axis_name="c", subcore_axis_name="s",
            num_cores=NC, num_subcores=NS),
        out_type=jax.ShapeDtypeStruct((NW, N_NODES), jnp.float32),
        scratch_types=[
            pltpu.VMEM((EPW,), jnp.int32),
            pltpu.VMEM((N_NODES,), jnp.float32),
        ],
        compiler_params=pltpu.CompilerParams(needs_layout_passes=False),
    )


def _tc_imp_body(x_ref, w1_ref, b1_ref, w2r_ref, b2_ref, imp_ref, mean_ref):
    x = x_ref[...]
    h1 = jnp.dot(x, w1_ref[...], preferred_element_type=jnp.float32)
    h1 = jnp.maximum(h1 + b1_ref[...], 0.0)
    imp_ref[...] = lax.dot_general(
        w2r_ref[...], h1, (((1,), (1,)), ((), ())),
        preferred_element_type=jnp.float32) + b2_ref[...]
    mean_ref[...] = jnp.sum(x, axis=0, keepdims=True) * (1.0 / N_NODES)


def _tc_body(x_ref, mask_ref, imp_ref, mean_ref,
             w0_ref, b0_ref, g0_ref, be0_ref, sel_ref, h_ref):
    x = x_ref[...]
    neg_inf = jnp.float32(-jnp.inf)
    impT = imp_ref[...]

    msum = jnp.sum(mask_ref[...], axis=0, keepdims=True)
    score = jnp.where(msum > 0.0, impT, neg_inf)

    idxs = lax.broadcasted_iota(jnp.int32, (1, N_NODES), 1)
    avail = idxs >= 0
    sels = []
    for _ in range(TOP_K):
        cand = jnp.where(avail, score, neg_inf)
        m = jnp.max(cand)
        eq = (cand == m) & avail
        sel = jnp.min(jnp.where(eq, idxs, N_NODES))
        sels.append(sel)
        avail = avail & (idxs != sel)

    row16 = lax.broadcasted_iota(jnp.int32, (16, 1), 0)
    selcol = jnp.full((16, 1), -1, jnp.int32)
    for k in range(TOP_K):
        selcol = jnp.where(row16 == k, sels[k], selcol)

    coliota = lax.broadcasted_iota(jnp.int32, (16, N_NODES), 1)
    onehot = (coliota == selcol).astype(jnp.float32)
    tgt = jnp.dot(onehot, x, preferred_element_type=jnp.float32)

    src = jnp.broadcast_to(mean_ref[...], (16, HIDDEN))
    combined = jnp.concatenate([src, tgt], axis=1)

    z = jnp.dot(combined, w0_ref[...],
                preferred_element_type=jnp.float32) + b0_ref[...]
    mu = jnp.mean(z, axis=1, keepdims=True)
    var = jnp.mean((z - mu) ** 2, axis=1, keepdims=True)
    h = (z - mu) / jnp.sqrt(var + 1e-5) * g0_ref[...] + be0_ref[...]
    h_ref[...] = jnp.maximum(h, 0.0)

    r8 = lax.broadcasted_iota(jnp.int32, (8, 128), 0)
    c128 = lax.broadcasted_iota(jnp.int32, (8, 128), 1)
    selmat = jnp.zeros((8, 128), jnp.int32)
    for k in range(TOP_K):
        selmat = jnp.where((r8 == 0) & (c128 == k), sels[k], selmat)
    sel_ref[...] = selmat


_tc_imp_call = pl.pallas_call(
    _tc_imp_body,
    out_shape=[
        jax.ShapeDtypeStruct((1, N_NODES), jnp.float32),
        jax.ShapeDtypeStruct((1, HIDDEN), jnp.float32),
    ],
)

_tc_call = pl.pallas_call(
    _tc_body,
    out_shape=[
        jax.ShapeDtypeStruct((8, 128), jnp.int32),
        jax.ShapeDtypeStruct((16, HIDDEN), jnp.float32),
    ],
)


def kernel(x, edge_index, hop_W0, hop_b0, hop_g0, hop_be0,
           hop_W1, hop_b1, hop_g1, hop_be1, imp_W1, imp_b1, imp_W2, imp_b2):
    edge_dst = edge_index[1].astype(jnp.int32)
    xf = x.astype(jnp.float32)
    mask32 = _sc_mask()(edge_dst)
    impT, mean = _tc_imp_call(
        xf,
        imp_W1,
        imp_b1.reshape(1, -1),
        imp_W2.reshape(1, -1),
        imp_b2.reshape(1, 1),
    )
    sel_mat, h16 = _tc_call(
        xf,
        mask32,
        impT,
        mean,
        hop_W0,
        hop_b0.reshape(1, -1),
        hop_g0.reshape(1, -1),
        hop_be0.reshape(1, -1),
    )
    selected = sel_mat[0, :TOP_K]
    h = h16[:TOP_K]
    return (selected, h)


# speedup vs baseline: 42.0584x; 1.0285x over previous
import functools

import jax
import jax.numpy as jnp
from jax import lax
from jax.experimental import pallas as pl
from jax.experimental.pallas import tpu as pltpu
from jax.experimental.pallas import tpu_sc as plsc

N_NODES = 10000
HIDDEN = 128
TOP_K = 10
N_EDGES = 320000
NC = 2
NS = 16
NW = NC * NS
EPW = N_EDGES // NW


_UNROLL = 8


def _sc_mask_body(edge_hbm, out_hbm, idx_v, mask_v):
    wid = lax.axis_index("s") * NC + lax.axis_index("c")
    base = wid * EPW
    pltpu.sync_copy(edge_hbm.at[pl.ds(base, EPW)], idx_v)

    zeros16 = jnp.zeros((16,), jnp.float32)

    def zero_body(i, carry):
        for u in range(_UNROLL):
            mask_v[pl.ds((i * _UNROLL + u) * 16, 16)] = zeros16
        return carry

    lax.fori_loop(0, N_NODES // (16 * _UNROLL), zero_body, 0)
    mask_v[pl.ds(N_NODES - 16, 16)] = zeros16

    ones16 = jnp.ones((16,), jnp.float32)

    def scatter_body(i, carry):
        for u in range(_UNROLL):
            idx = idx_v[pl.ds((i * _UNROLL + u) * 16, 16)]
            plsc.store_scatter(mask_v, [idx], ones16)
        return carry

    lax.fori_loop(0, EPW // (16 * _UNROLL), scatter_body, 0)
    idx = idx_v[pl.ds(EPW - 16, 16)]
    plsc.store_scatter(mask_v, [idx], ones16)
    pltpu.sync_copy(mask_v, out_hbm.at[wid])


@functools.cache
def _sc_mask():
    return pl.kernel(
        _sc_mask_body,
        mesh=plsc.VectorSubcoreMesh(
            core_axis_name="c", subcore_axis_name="s",
            num_cores=NC, num_subcores=NS),
        out_type=jax.ShapeDtypeStruct((NW, N_NODES), jnp.float32),
        scratch_types=[
            pltpu.VMEM((EPW,), jnp.int32),
            pltpu.VMEM((N_NODES,), jnp.float32),
        ],
        compiler_params=pltpu.CompilerParams(
            needs_layout_passes=False,
            skip_device_barrier=True,
            disable_semaphore_checks=True,
        ),
    )


def _tc_imp_body(x_ref, w1_ref, b1_ref, w2r_ref, b2_ref, imp_ref, mean_ref):
    x = x_ref[...]
    h1 = jnp.dot(x, w1_ref[...], preferred_element_type=jnp.float32)
    h1 = jnp.maximum(h1 + b1_ref[...], 0.0)
    imp_ref[...] = lax.dot_general(
        w2r_ref[...], h1, (((1,), (1,)), ((), ())),
        preferred_element_type=jnp.float32) + b2_ref[...]
    mean_ref[...] = jnp.sum(x, axis=0, keepdims=True) * (1.0 / N_NODES)


def _tc_body(x_ref, mask_ref, imp_ref, mean_ref,
             w0_ref, b0_ref, g0_ref, be0_ref, sel_ref, h_ref,
             tgt_v, sem):
    neg_inf = jnp.float32(-jnp.inf)
    impT = imp_ref[...]

    msum = jnp.sum(mask_ref[...], axis=0, keepdims=True)
    score = jnp.where(msum > 0.0, impT, neg_inf)

    idxs = lax.broadcasted_iota(jnp.int32, (1, N_NODES), 1)
    avail = idxs >= 0
    sels = []
    for _ in range(TOP_K):
        cand = jnp.where(avail, score, neg_inf)
        m = jnp.max(cand)
        eq = (cand == m) & avail
        sel = jnp.min(jnp.where(eq, idxs, N_NODES))
        sels.append(sel)
        avail = avail & (idxs != sel)

    tgt_v[pl.ds(8, 8), :] = jnp.zeros((8, 128), jnp.float32)
    copies = [
        pltpu.make_async_copy(
            x_ref.at[pl.ds(sels[k], 1), :], tgt_v.at[pl.ds(k, 1), :], sem)
        for k in range(TOP_K)
    ]
    for c in copies:
        c.start()
    for c in copies:
        c.wait()
    tgt = tgt_v[...]

    src = jnp.broadcast_to(mean_ref[...], (16, HIDDEN))
    combined = jnp.concatenate([src, tgt], axis=1)

    z = jnp.dot(combined, w0_ref[...],
                preferred_element_type=jnp.float32) + b0_ref[...]
    mu = jnp.mean(z, axis=1, keepdims=True)
    var = jnp.mean((z - mu) ** 2, axis=1, keepdims=True)
    h = (z - mu) / jnp.sqrt(var + 1e-5) * g0_ref[...] + be0_ref[...]
    h_ref[...] = jnp.maximum(h, 0.0)

    r8 = lax.broadcasted_iota(jnp.int32, (8, 128), 0)
    c128 = lax.broadcasted_iota(jnp.int32, (8, 128), 1)
    selmat = jnp.zeros((8, 128), jnp.int32)
    for k in range(TOP_K):
        selmat = jnp.where((r8 == 0) & (c128 == k), sels[k], selmat)
    sel_ref[...] = selmat


_tc_imp_call = pl.pallas_call(
    _tc_imp_body,
    out_shape=[
        jax.ShapeDtypeStruct((1, N_NODES), jnp.float32),
        jax.ShapeDtypeStruct((1, HIDDEN), jnp.float32),
    ],
)

_tc_call = pl.pallas_call(
    _tc_body,
    in_specs=[
        pl.BlockSpec(memory_space=pl.ANY),
    ] + [pl.BlockSpec()] * 7 + [
    ],
    out_shape=[
        jax.ShapeDtypeStruct((8, 128), jnp.int32),
        jax.ShapeDtypeStruct((16, HIDDEN), jnp.float32),
    ],
    scratch_shapes=[
        pltpu.VMEM((16, HIDDEN), jnp.float32),
        pltpu.SemaphoreType.DMA,
    ],
)


def kernel(x, edge_index, hop_W0, hop_b0, hop_g0, hop_be0,
           hop_W1, hop_b1, hop_g1, hop_be1, imp_W1, imp_b1, imp_W2, imp_b2):
    edge_dst = edge_index[1].astype(jnp.int32)
    xf = x.astype(jnp.float32)
    mask32 = _sc_mask()(edge_dst)
    impT, mean = _tc_imp_call(
        xf,
        imp_W1,
        imp_b1.reshape(1, -1),
        imp_W2.reshape(1, -1),
        imp_b2.reshape(1, 1),
    )
    sel_mat, h16 = _tc_call(
        xf,
        mask32,
        impT,
        mean,
        hop_W0,
        hop_b0.reshape(1, -1),
        hop_g0.reshape(1, -1),
        hop_be0.reshape(1, -1),
    )
    selected = sel_mat[0, :TOP_K]
    h = h16[:TOP_K]
    return (selected, h)


# speedup vs baseline: 53.9801x; 1.2835x over previous
import functools

import jax
import jax.numpy as jnp
from jax import lax
from jax.experimental import pallas as pl
from jax.experimental.pallas import tpu as pltpu
from jax.experimental.pallas import tpu_sc as plsc

N_NODES = 10000
HIDDEN = 128
TOP_K = 10
N_EDGES = 320000
NC = 2
NS = 16
NW = NC * NS
EPW = N_EDGES // NW


_UNROLL = 8


def _sc_mask_body(edge_hbm, out_hbm, idx_v, mask_v):
    wid = lax.axis_index("s") * NC + lax.axis_index("c")
    base = wid * EPW
    pltpu.sync_copy(edge_hbm.at[pl.ds(N_EDGES + base, EPW)], idx_v)

    zeros16 = jnp.zeros((16,), jnp.float32)

    def zero_body(i, carry):
        for u in range(_UNROLL):
            mask_v[pl.ds((i * _UNROLL + u) * 16, 16)] = zeros16
        return carry

    lax.fori_loop(0, N_NODES // (16 * _UNROLL), zero_body, 0)
    mask_v[pl.ds(N_NODES - 16, 16)] = zeros16

    ones16 = jnp.ones((16,), jnp.float32)

    def scatter_body(i, carry):
        for u in range(_UNROLL):
            idx = idx_v[pl.ds((i * _UNROLL + u) * 16, 16)]
            plsc.store_scatter(mask_v, [idx], ones16)
        return carry

    lax.fori_loop(0, EPW // (16 * _UNROLL), scatter_body, 0)
    idx = idx_v[pl.ds(EPW - 16, 16)]
    plsc.store_scatter(mask_v, [idx], ones16)
    pltpu.sync_copy(mask_v, out_hbm.at[wid])


@functools.cache
def _sc_mask():
    return pl.kernel(
        _sc_mask_body,
        mesh=plsc.VectorSubcoreMesh(
            core_axis_name="c", subcore_axis_name="s",
            num_cores=NC, num_subcores=NS),
        out_type=jax.ShapeDtypeStruct((NW, N_NODES), jnp.float32),
        scratch_types=[
            pltpu.VMEM((EPW,), jnp.int32),
            pltpu.VMEM((N_NODES,), jnp.float32),
        ],
        compiler_params=pltpu.CompilerParams(needs_layout_passes=False),
    )


def _tc_imp_body(x_ref, w1_ref, b1_ref, w2r_ref, b2_ref, imp_ref, mean_ref):
    x = x_ref[...]
    h1 = jnp.dot(x, w1_ref[...], preferred_element_type=jnp.float32)
    h1 = jnp.maximum(h1 + b1_ref[...], 0.0)
    imp_ref[...] = lax.dot_general(
        w2r_ref[...], h1, (((1,), (1,)), ((), ())),
        preferred_element_type=jnp.float32) + b2_ref[...]
    mean_ref[...] = jnp.sum(x, axis=0, keepdims=True) * (1.0 / N_NODES)


def _tc_body(x_ref, mask_ref, imp_ref, mean_ref,
             w0_ref, b0_ref, g0_ref, be0_ref, sel_ref, h_ref,
             tgt_v, sem):
    neg_inf = jnp.float32(-jnp.inf)
    impT = imp_ref[...]

    msum = jnp.sum(mask_ref[...], axis=0, keepdims=True)
    score = jnp.where(msum > 0.0, impT, neg_inf)

    idxs = lax.broadcasted_iota(jnp.int32, (1, N_NODES), 1)
    avail = idxs >= 0
    sels = []
    for _ in range(TOP_K):
        cand = jnp.where(avail, score, neg_inf)
        m = jnp.max(cand)
        eq = (cand == m) & avail
        sel = jnp.min(jnp.where(eq, idxs, N_NODES))
        sels.append(sel)
        avail = avail & (idxs != sel)

    tgt_v[pl.ds(8, 8), :] = jnp.zeros((8, 128), jnp.float32)
    copies = [
        pltpu.make_async_copy(
            x_ref.at[pl.ds(sels[k], 1), :], tgt_v.at[pl.ds(k, 1), :], sem)
        for k in range(TOP_K)
    ]
    for c in copies:
        c.start()
    for c in copies:
        c.wait()
    tgt = tgt_v[...]

    src = jnp.broadcast_to(mean_ref[...], (16, HIDDEN))
    combined = jnp.concatenate([src, tgt], axis=1)

    z = jnp.dot(combined, w0_ref[...],
                preferred_element_type=jnp.float32) + b0_ref[...]
    mu = jnp.mean(z, axis=1, keepdims=True)
    var = jnp.mean((z - mu) ** 2, axis=1, keepdims=True)
    h = (z - mu) / jnp.sqrt(var + 1e-5) * g0_ref[...] + be0_ref[...]
    h_ref[...] = jnp.maximum(h, 0.0)

    r8 = lax.broadcasted_iota(jnp.int32, (8, 128), 0)
    c128 = lax.broadcasted_iota(jnp.int32, (8, 128), 1)
    selmat = jnp.zeros((8, 128), jnp.int32)
    for k in range(TOP_K):
        selmat = jnp.where((r8 == 0) & (c128 == k), sels[k], selmat)
    sel_ref[...] = selmat


_tc_imp_call = pl.pallas_call(
    _tc_imp_body,
    out_shape=[
        jax.ShapeDtypeStruct((1, N_NODES), jnp.float32),
        jax.ShapeDtypeStruct((1, HIDDEN), jnp.float32),
    ],
)

_tc_call = pl.pallas_call(
    _tc_body,
    in_specs=[
        pl.BlockSpec(memory_space=pl.ANY),
    ] + [pl.BlockSpec()] * 7 + [
    ],
    out_shape=[
        jax.ShapeDtypeStruct((8, 128), jnp.int32),
        jax.ShapeDtypeStruct((16, HIDDEN), jnp.float32),
    ],
    scratch_shapes=[
        pltpu.VMEM((16, HIDDEN), jnp.float32),
        pltpu.SemaphoreType.DMA,
    ],
)


def kernel(x, edge_index, hop_W0, hop_b0, hop_g0, hop_be0,
           hop_W1, hop_b1, hop_g1, hop_be1, imp_W1, imp_b1, imp_W2, imp_b2):
    xf = x.astype(jnp.float32)
    mask32 = _sc_mask()(edge_index.astype(jnp.int32).reshape(2 * N_EDGES))
    impT, mean = _tc_imp_call(
        xf,
        imp_W1,
        imp_b1.reshape(1, -1),
        imp_W2.reshape(1, -1),
        imp_b2.reshape(1, 1),
    )
    sel_mat, h16 = _tc_call(
        xf,
        mask32,
        impT,
        mean,
        hop_W0,
        hop_b0.reshape(1, -1),
        hop_g0.reshape(1, -1),
        hop_be0.reshape(1, -1),
    )
    selected = sel_mat[0, :TOP_K]
    h = h16[:TOP_K]
    return (selected, h)
